# baseline (speedup 1.0000x reference)
import functools

import jax
import jax.numpy as jnp
from jax import lax
from jax.experimental import pallas as pl
from jax.experimental.pallas import tpu as pltpu
from jax.experimental.pallas import tpu_sc as plsc

_BATCH = 16384
_CLASSES = 1000
_KEEP = int(_BATCH * 0.7)

_SC_ROWS = 7168
_TC_ROWS = _BATCH - _SC_ROWS
_ROWS = 1024

_NW = 32
_RPW = _SC_ROWS // _NW
_CHUNK = 16
_NCHUNK = _RPW // _CHUNK
_PITCH = _CLASSES


_NFULL = 992 // 16
_TAIL = 984


def _sc_ce(pred_hbm, tgt_hbm, m_hbm, s_hbm, tv_hbm,
           buf0, buf1, tgtbuf, mbuf, sbuf, tvbuf, sem0, sem1):
    wid = lax.axis_index("s") * 2 + lax.axis_index("c")
    base = wid * _RPW
    pltpu.sync_copy(tgt_hbm.at[pl.ds(base, _RPW)], tgtbuf)
    lanes = lax.iota(jnp.int32, 16)
    lanes_f = lanes.astype(jnp.float32)
    neginf = jnp.full((16,), -jnp.inf, jnp.float32)
    tail_valid = lanes >= 8

    pltpu.async_copy(pred_hbm.at[pl.ds(base, _CHUNK), :], buf0, sem0)
    pltpu.async_copy(pred_hbm.at[pl.ds(base + _CHUNK, _CHUNK), :], buf1, sem1)

    def process(ci, cur):
        tvec = tgtbuf[pl.ds(ci * _CHUNK, 16)]
        safe = jnp.clip(tvec, 0, _CLASSES - 1).astype(jnp.float32)
        t_rs = [jnp.max(jnp.where(lanes == r, safe, -1.0))
                for r in range(_CHUNK)]

        def mb(j, accs):
            return tuple(
                jnp.maximum(accs[r], cur[r, pl.ds(j * 16, 16)])
                for r in range(_CHUNK))

        maccs = lax.fori_loop(0, _NFULL, mb, (neginf,) * _CHUNK, unroll=2)
        gts = [cur[r, pl.ds(_TAIL, 16)] for r in range(_CHUNK)]
        m_rs = [jnp.max(jnp.maximum(maccs[r], gts[r])) for r in range(_CHUNK)]

        def sb(j, c):
            saccs, tvaccs = c
            colid = (j * 16).astype(jnp.float32) + lanes_f
            sn, tn = [], []
            for r in range(_CHUNK):
                g = cur[r, pl.ds(j * 16, 16)]
                sn.append(saccs[r] + jnp.exp(g - m_rs[r]))
                tn.append(jnp.where(colid == t_rs[r], g, tvaccs[r]))
            return tuple(sn), tuple(tn)

        zero16 = jnp.zeros((16,), jnp.float32)
        saccs, tvaccs = lax.fori_loop(
            0, _NFULL, sb, ((zero16,) * _CHUNK, (neginf,) * _CHUNK),
            unroll=2)

        m_vec = zero16
        s_vec = zero16
        tv_vec = zero16
        tailcol = _TAIL + lanes_f
        for r in range(_CHUNK):
            sacc = saccs[r] + jnp.where(
                tail_valid, jnp.exp(gts[r] - m_rs[r]), 0.0)
            tvacc = jnp.where(
                jnp.logical_and(tailcol == t_rs[r], tail_valid),
                gts[r], tvaccs[r])
            m_vec = jnp.where(lanes == r, m_rs[r], m_vec)
            s_vec = jnp.where(lanes == r, jnp.sum(sacc), s_vec)
            tv_vec = jnp.where(lanes == r, jnp.max(tvacc), tv_vec)

        mbuf[pl.ds(ci * _CHUNK, 16)] = m_vec
        sbuf[pl.ds(ci * _CHUNK, 16)] = s_vec
        tvbuf[pl.ds(ci * _CHUNK, 16)] = tv_vec

    def chunk_body(i, carry):
        ci0 = 2 * i
        ci1 = 2 * i + 1
        pltpu.make_async_copy(
            pred_hbm.at[pl.ds(base, _CHUNK), :], buf0, sem0).wait()
        process(ci0, buf0)

        @pl.when(ci0 + 2 < _NCHUNK)
        def _():
            pltpu.async_copy(
                pred_hbm.at[pl.ds(base + (ci0 + 2) * _CHUNK, _CHUNK), :],
                buf0, sem0)

        pltpu.make_async_copy(
            pred_hbm.at[pl.ds(base, _CHUNK), :], buf1, sem1).wait()
        process(ci1, buf1)

        @pl.when(ci1 + 2 < _NCHUNK)
        def _():
            pltpu.async_copy(
                pred_hbm.at[pl.ds(base + (ci1 + 2) * _CHUNK, _CHUNK), :],
                buf1, sem1)

        return carry

    lax.fori_loop(0, _NCHUNK // 2, chunk_body, jnp.int32(0))

    pltpu.sync_copy(mbuf, m_hbm.at[pl.ds(base, _RPW)])
    pltpu.sync_copy(sbuf, s_hbm.at[pl.ds(base, _RPW)])
    pltpu.sync_copy(tvbuf, tv_hbm.at[pl.ds(base, _RPW)])


_sc_ce_call = functools.partial(
    pl.kernel,
    out_type=[
        jax.ShapeDtypeStruct((_SC_ROWS,), jnp.float32),
        jax.ShapeDtypeStruct((_SC_ROWS,), jnp.float32),
        jax.ShapeDtypeStruct((_SC_ROWS,), jnp.float32),
    ],
    mesh=plsc.VectorSubcoreMesh(core_axis_name="c", subcore_axis_name="s"),
    compiler_params=pltpu.CompilerParams(
        needs_layout_passes=False, use_tc_tiling_on_sc=True),
    scratch_types=[
        pltpu.VMEM((_CHUNK, _CLASSES), jnp.float32),
        pltpu.VMEM((_CHUNK, _CLASSES), jnp.float32),
        pltpu.VMEM((_RPW,), jnp.int32),
        pltpu.VMEM((_RPW,), jnp.float32),
        pltpu.VMEM((_RPW,), jnp.float32),
        pltpu.VMEM((_RPW,), jnp.float32),
        pltpu.SemaphoreType.DMA,
        pltpu.SemaphoreType.DMA,
    ],
)(_sc_ce)


def _ce_kernel(pred_ref, tgt_ref, out_ref):
    x = pred_ref[...]
    tgt = tgt_ref[...]
    m = jnp.max(x, axis=1, keepdims=True)
    s = jnp.sum(jnp.exp(x - m), axis=1, keepdims=True)
    lse = m + jnp.log(s)
    col = jax.lax.broadcasted_iota(jnp.int32, x.shape, 1)
    safe = jnp.clip(tgt, 0, _CLASSES - 1)
    tsel = jnp.sum(jnp.where(col == safe, x, 0.0), axis=1, keepdims=True)
    ce = lse - tsel
    ce = jnp.where(tgt == -1, 0.0, ce)
    out_ref[...] = ce


def _topk_kernel(m_ref, s_ref, tv_ref, tgt_ref, ce_tc_ref, out_ref):
    ce_sc = m_ref[...] + jnp.log(s_ref[...]) - tv_ref[...]
    ce_sc = jnp.where(tgt_ref[...] == -1, 0.0, ce_sc)
    ce_tc = ce_tc_ref[...]

    def count_ge(t):
        n = jnp.sum((ce_sc >= t).astype(jnp.int32))
        if _TC_ROWS:
            n = n + jnp.sum((ce_tc >= t).astype(jnp.int32))
        return n

    def body(_, lohi):
        lo, hi = lohi
        mid = lo + (hi - lo) // 2
        t = jax.lax.bitcast_convert_type(mid, jnp.float32)
        ge = count_ge(t) >= _KEEP
        return jnp.where(ge, mid, lo), jnp.where(ge, hi, mid)

    lo, _ = jax.lax.fori_loop(
        0, 32, body, (jnp.int32(0), jnp.int32(0x7F800000))
    )
    t = jax.lax.bitcast_convert_type(lo, jnp.float32)
    cnt_gt = jnp.sum((ce_sc > t).astype(jnp.int32))
    sum_gt = jnp.sum(jnp.where(ce_sc > t, ce_sc, 0.0))
    if _TC_ROWS:
        cnt_gt = cnt_gt + jnp.sum((ce_tc > t).astype(jnp.int32))
        sum_gt = sum_gt + jnp.sum(jnp.where(ce_tc > t, ce_tc, 0.0))
    total = sum_gt + (_KEEP - cnt_gt).astype(jnp.float32) * t
    out_ref[...] = jnp.broadcast_to(total / jnp.float32(_KEEP), (1, 1))


@jax.jit
def kernel(pred, target):
    tgt = target.astype(jnp.int32)
    m, s, tv = _sc_ce_call(pred, tgt)

    if _TC_ROWS:
        grid = _TC_ROWS // _ROWS
        ce_tc = pl.pallas_call(
            _ce_kernel,
            grid=(grid,),
            in_specs=[
                pl.BlockSpec((_ROWS, _CLASSES),
                             lambda i: (i + _SC_ROWS // _ROWS, 0)),
                pl.BlockSpec((_ROWS, 1),
                             lambda i: (i + _SC_ROWS // _ROWS, 0)),
            ],
            out_specs=pl.BlockSpec((_ROWS, 1), lambda i: (i, 0)),
            out_shape=jax.ShapeDtypeStruct((_TC_ROWS, 1), jnp.float32),
            compiler_params=pltpu.CompilerParams(
                dimension_semantics=("arbitrary",),
            ),
        )(pred, tgt.reshape(_BATCH, 1))
        ce_tc2 = ce_tc.reshape(_TC_ROWS // 128, 128)
    else:
        ce_tc2 = jnp.zeros((8, 128), jnp.float32)

    sc_r = _SC_ROWS // 128
    out = pl.pallas_call(
        _topk_kernel,
        out_shape=jax.ShapeDtypeStruct((1, 1), jnp.float32),
    )(m.reshape(sc_r, 128), s.reshape(sc_r, 128), tv.reshape(sc_r, 128),
      tgt[:_SC_ROWS].reshape(sc_r, 128), ce_tc2)
    return out[0, 0]


# speedup vs baseline: 3.9209x; 3.9209x over previous
import jax
import jax.numpy as jnp
from jax.experimental import pallas as pl
from jax.experimental.pallas import tpu as pltpu

_BATCH = 16384
_CLASSES = 1000
_KEEP = int(_BATCH * 0.7)
_COLS = 4096


def _ce_t_kernel(predt_ref, tgt_ref, out_ref):
    x = predt_ref[...]
    tgt = tgt_ref[...]
    m = jnp.max(x, axis=0, keepdims=True)
    s = jnp.sum(jnp.exp(x - m), axis=0, keepdims=True)
    lse = m + jnp.log(s)
    row = jax.lax.broadcasted_iota(jnp.int32, x.shape, 0)
    safe = jnp.clip(tgt, 0, _CLASSES - 1)
    tsel = jnp.sum(jnp.where(row == safe, x, 0.0), axis=0, keepdims=True)
    ce = lse - tsel
    ce = jnp.where(tgt == -1, 0.0, ce)
    out_ref[...] = ce


def _topk_kernel(ce_ref, out_ref):
    ce = ce_ref[...]

    def body(_, lohi):
        lo, hi = lohi
        mid = lo + (hi - lo) // 2
        t = jax.lax.bitcast_convert_type(mid, jnp.float32)
        cnt = jnp.sum((ce >= t).astype(jnp.int32))
        ge = cnt >= _KEEP
        return jnp.where(ge, mid, lo), jnp.where(ge, hi, mid)

    lo, _ = jax.lax.fori_loop(
        0, 32, body, (jnp.int32(0), jnp.int32(0x7F800000))
    )
    t = jax.lax.bitcast_convert_type(lo, jnp.float32)
    gt = ce > t
    cnt_gt = jnp.sum(gt.astype(jnp.int32))
    sum_gt = jnp.sum(jnp.where(gt, ce, 0.0))
    total = sum_gt + (_KEEP - cnt_gt).astype(jnp.float32) * t
    out_ref[...] = jnp.broadcast_to(total / jnp.float32(_KEEP), (1, 1))


@jax.jit
def kernel(pred, target):
    predt = pred.T
    tgt = target.astype(jnp.int32).reshape(1, _BATCH)
    grid = _BATCH // _COLS
    ce = pl.pallas_call(
        _ce_t_kernel,
        grid=(grid,),
        in_specs=[
            pl.BlockSpec((_CLASSES, _COLS), lambda i: (0, i)),
            pl.BlockSpec((1, _COLS), lambda i: (0, i)),
        ],
        out_specs=pl.BlockSpec((1, _COLS), lambda i: (0, i)),
        out_shape=jax.ShapeDtypeStruct((1, _BATCH), jnp.float32),
        compiler_params=pltpu.CompilerParams(
            dimension_semantics=("arbitrary",),
        ),
    )(predt, tgt)

    out = pl.pallas_call(
        _topk_kernel,
        out_shape=jax.ShapeDtypeStruct((1, 1), jnp.float32),
    )(ce)
    return out[0, 0]


# speedup vs baseline: 3.9952x; 1.0189x over previous
import jax
import jax.numpy as jnp
from jax.experimental import pallas as pl
from jax.experimental.pallas import tpu as pltpu

_BATCH = 16384
_CLASSES = 1000
_KEEP = int(_BATCH * 0.7)
_COLS = 4096


def _ce_t_kernel(predt_ref, tgt_ref, out_ref):
    x = predt_ref[...]
    tgt = tgt_ref[...]
    m = jnp.max(x, axis=0, keepdims=True)
    s = jnp.sum(jnp.exp(x - m), axis=0, keepdims=True)
    lse = m + jnp.log(s)
    row = jax.lax.broadcasted_iota(jnp.int32, x.shape, 0)
    safe = jnp.clip(tgt, 0, _CLASSES - 1)
    tsel = jnp.sum(jnp.where(row == safe, x, 0.0), axis=0, keepdims=True)
    ce = lse - tsel
    ce = jnp.where(tgt == -1, 0.0, ce)
    out_ref[...] = ce


def _topk_kernel(ce_ref, out_ref):
    ce = ce_ref[...]

    def body(_, lohi):
        lo, hi = lohi
        w = hi - lo
        m1 = lo + w // 4
        m2 = lo + w // 2
        m3 = m2 + w // 4
        c1 = jnp.sum((ce >= jax.lax.bitcast_convert_type(m1, jnp.float32))
                     .astype(jnp.int32))
        c2 = jnp.sum((ce >= jax.lax.bitcast_convert_type(m2, jnp.float32))
                     .astype(jnp.int32))
        c3 = jnp.sum((ce >= jax.lax.bitcast_convert_type(m3, jnp.float32))
                     .astype(jnp.int32))
        ge1 = c1 >= _KEEP
        ge2 = c2 >= _KEEP
        ge3 = c3 >= _KEEP
        lo2 = jnp.where(ge3, m3, jnp.where(ge2, m2, jnp.where(ge1, m1, lo)))
        hi2 = jnp.where(jnp.logical_not(ge1), m1,
                        jnp.where(jnp.logical_not(ge2), m2,
                                  jnp.where(jnp.logical_not(ge3), m3, hi)))
        return lo2, hi2

    lo, _ = jax.lax.fori_loop(
        0, 18, body, (jnp.int32(0), jnp.int32(0x7F800000))
    )
    t = jax.lax.bitcast_convert_type(lo, jnp.float32)
    gt = ce > t
    cnt_gt = jnp.sum(gt.astype(jnp.int32))
    sum_gt = jnp.sum(jnp.where(gt, ce, 0.0))
    total = sum_gt + (_KEEP - cnt_gt).astype(jnp.float32) * t
    out_ref[...] = jnp.broadcast_to(total / jnp.float32(_KEEP), (1, 1))


@jax.jit
def kernel(pred, target):
    predt = pred.T
    tgt = target.astype(jnp.int32).reshape(1, _BATCH)
    grid = _BATCH // _COLS
    ce = pl.pallas_call(
        _ce_t_kernel,
        grid=(grid,),
        in_specs=[
            pl.BlockSpec((_CLASSES, _COLS), lambda i: (0, i)),
            pl.BlockSpec((1, _COLS), lambda i: (0, i)),
        ],
        out_specs=pl.BlockSpec((1, _COLS), lambda i: (0, i)),
        out_shape=jax.ShapeDtypeStruct((1, _BATCH), jnp.float32),
        compiler_params=pltpu.CompilerParams(
            dimension_semantics=("arbitrary",),
        ),
    )(predt, tgt)

    out = pl.pallas_call(
        _topk_kernel,
        out_shape=jax.ShapeDtypeStruct((1, 1), jnp.float32),
    )(ce)
    return out[0, 0]


# speedup vs baseline: 4.0786x; 1.0209x over previous
import jax
import jax.numpy as jnp
from jax.experimental import pallas as pl
from jax.experimental.pallas import tpu as pltpu

_BATCH = 16384
_CLASSES = 1000
_KEEP = int(_BATCH * 0.7)
_COLS = 2048


def _ce_t_kernel(predt_ref, tgt_ref, out_ref):
    x = predt_ref[...]
    tgt = tgt_ref[...]
    m = jnp.max(x, axis=0, keepdims=True)
    s = jnp.sum(jnp.exp(x - m), axis=0, keepdims=True)
    lse = m + jnp.log(s)
    row = jax.lax.broadcasted_iota(jnp.int32, x.shape, 0)
    safe = jnp.clip(tgt, 0, _CLASSES - 1)
    tsel = jnp.sum(jnp.where(row == safe, x, 0.0), axis=0, keepdims=True)
    ce = lse - tsel
    ce = jnp.where(tgt == -1, 0.0, ce)
    out_ref[...] = ce


def _topk_kernel(ce_ref, out_ref):
    ce = ce_ref[...]

    def body(_, lohi):
        lo, hi = lohi
        w = hi - lo
        m1 = lo + w // 4
        m2 = lo + w // 2
        m3 = m2 + w // 4
        c1 = jnp.sum((ce >= jax.lax.bitcast_convert_type(m1, jnp.float32))
                     .astype(jnp.int32))
        c2 = jnp.sum((ce >= jax.lax.bitcast_convert_type(m2, jnp.float32))
                     .astype(jnp.int32))
        c3 = jnp.sum((ce >= jax.lax.bitcast_convert_type(m3, jnp.float32))
                     .astype(jnp.int32))
        ge1 = c1 >= _KEEP
        ge2 = c2 >= _KEEP
        ge3 = c3 >= _KEEP
        lo2 = jnp.where(ge3, m3, jnp.where(ge2, m2, jnp.where(ge1, m1, lo)))
        hi2 = jnp.where(jnp.logical_not(ge1), m1,
                        jnp.where(jnp.logical_not(ge2), m2,
                                  jnp.where(jnp.logical_not(ge3), m3, hi)))
        return lo2, hi2

    lo, _ = jax.lax.fori_loop(
        0, 18, body, (jnp.int32(0), jnp.int32(0x7F800000))
    )
    t = jax.lax.bitcast_convert_type(lo, jnp.float32)
    gt = ce > t
    cnt_gt = jnp.sum(gt.astype(jnp.int32))
    sum_gt = jnp.sum(jnp.where(gt, ce, 0.0))
    total = sum_gt + (_KEEP - cnt_gt).astype(jnp.float32) * t
    out_ref[...] = jnp.broadcast_to(total / jnp.float32(_KEEP), (1, 1))


@jax.jit
def kernel(pred, target):
    predt = pred.T
    tgt = target.astype(jnp.int32).reshape(1, _BATCH)
    grid = _BATCH // _COLS
    ce = pl.pallas_call(
        _ce_t_kernel,
        grid=(grid,),
        in_specs=[
            pl.BlockSpec((_CLASSES, _COLS), lambda i: (0, i)),
            pl.BlockSpec((1, _COLS), lambda i: (0, i)),
        ],
        out_specs=pl.BlockSpec((1, _COLS), lambda i: (0, i)),
        out_shape=jax.ShapeDtypeStruct((1, _BATCH), jnp.float32),
        compiler_params=pltpu.CompilerParams(
            dimension_semantics=("arbitrary",),
        ),
    )(predt, tgt)

    out = pl.pallas_call(
        _topk_kernel,
        out_shape=jax.ShapeDtypeStruct((1, 1), jnp.float32),
    )(ce)
    return out[0, 0]
